# Initial kernel scaffold; baseline (speedup 1.0000x reference)
import functools

import jax
import jax.numpy as jnp
from jax import lax
from jax.experimental import pallas as pl
from jax.experimental.pallas import tpu as pltpu
from jax.experimental.pallas import tpu_sc as plsc

N_NODES = 10000
N_EDGES = 160000
D = 256
HALF = 5000
ACC_ROWS = 5120
DUMMY = ACC_ROWS - 1
NS = 16
ET = N_EDGES // NS
K = 80
CHUNKS = ET // K
RPT = ACC_ROWS // NS


def _sc_body(x_hbm, row_hbm, col_hbm, s_out, c_out,
             row_v, col_v, loc_v, rows_v, ones_v, zrow_v, zcnt_v,
             s_acc, c_acc, sem):
    c = lax.axis_index("c")
    s = lax.axis_index("s")

    def _zi(i, _):
        zrow_v[i // 16, pl.ds((i % 16) * 16, 16)] = jnp.zeros((16,), jnp.float32)
        return 0
    lax.fori_loop(0, 16 * (D // 16), _zi, 0)

    def _oi(i, _):
        ones_v[i, :] = jnp.ones((16,), jnp.float32)
        return 0
    lax.fori_loop(0, K, _oi, 0)

    def _zc(i, _):
        zcnt_v[i, :] = jnp.zeros((16,), jnp.float32)
        return 0
    lax.fori_loop(0, RPT, _zc, 0)

    def _zs(r, _):
        pltpu.sync_copy(zrow_v, s_acc.at[pl.ds(RPT * s + 16 * r, 16)])
        return 0
    lax.fori_loop(0, RPT // 16, _zs, 0)
    pltpu.sync_copy(zcnt_v, c_acc.at[pl.ds(RPT * s, RPT)])
    plsc.subcore_barrier()

    lo = c * HALF

    def _chunk(j, _):
        base = s * ET + j * K
        pltpu.sync_copy(row_hbm.at[pl.ds(base, K)], row_v)
        pltpu.sync_copy(col_hbm.at[pl.ds(base, K)], col_v)
        for t in range(K // 16):
            r16 = row_v[pl.ds(t * 16, 16)]
            l16 = r16 - lo
            m = (l16 >= 0) & (l16 < HALF)
            loc_v[pl.ds(t * 16, 16)] = jnp.where(m, l16, DUMMY)
        pltpu.async_copy(x_hbm.at[col_v], rows_v, sem).wait()
        pltpu.sync_copy(rows_v, s_acc.at[loc_v], add=True)
        pltpu.sync_copy(ones_v, c_acc.at[loc_v], add=True)
        return 0
    lax.fori_loop(0, CHUNKS, _chunk, 0)

    plsc.subcore_barrier()
    pltpu.sync_copy(s_acc.at[pl.ds(RPT * s, RPT)], s_out.at[c, pl.ds(RPT * s, RPT)])
    pltpu.sync_copy(c_acc.at[pl.ds(RPT * s, RPT)], c_out.at[c, pl.ds(RPT * s, RPT)])


def _sc_segment_sum(x, row, col):
    mesh = plsc.VectorSubcoreMesh(core_axis_name="c", subcore_axis_name="s")
    kern = functools.partial(
        pl.kernel,
        mesh=mesh,
        out_type=[
            jax.ShapeDtypeStruct((2, ACC_ROWS, D), jnp.float32),
            jax.ShapeDtypeStruct((2, ACC_ROWS, 16), jnp.float32),
        ],
        scratch_types=[
            pltpu.VMEM((K,), jnp.int32),
            pltpu.VMEM((K,), jnp.int32),
            pltpu.VMEM((K,), jnp.int32),
            pltpu.VMEM((K, D), jnp.float32),
            pltpu.VMEM((K, 16), jnp.float32),
            pltpu.VMEM((16, D), jnp.float32),
            pltpu.VMEM((RPT, 16), jnp.float32),
            pltpu.VMEM_SHARED((ACC_ROWS, D), jnp.float32),
            pltpu.VMEM_SHARED((ACC_ROWS, 16), jnp.float32),
            pltpu.SemaphoreType.DMA,
        ],
    )(_sc_body)
    return kern(x, row, col)


def _tc_body(x_ref, s_ref, cnt_ref, wlin_ref, blin_ref, wmsg_ref, bmsg_ref,
             wupd_ref, bupd_ref, o_ref):
    xb = x_ref[...]
    hs = jnp.dot(xb, wlin_ref[...], preferred_element_type=jnp.float32) + blin_ref[...]
    cnt = cnt_ref[0, :, 0:1]
    m = s_ref[0] / jnp.maximum(cnt, 1.0)
    hm = jnp.dot(m, wmsg_ref[...], preferred_element_type=jnp.float32) + bmsg_ref[...]
    ha = jnp.where(cnt > 0.0, hm, 0.0)
    h = (jnp.dot(hs, wupd_ref[0:D], preferred_element_type=jnp.float32)
         + jnp.dot(ha, wupd_ref[D:2 * D], preferred_element_type=jnp.float32)
         + bupd_ref[...])
    o_ref[...] = jnp.maximum(h, 0.0)


def _tc_dense(x, s2, c2, W_lin, b_lin, W_msg, b_msg, W_upd, b_upd):
    B = 1000
    nbh = HALF // B
    return pl.pallas_call(
        _tc_body,
        grid=(N_NODES // B,),
        in_specs=[
            pl.BlockSpec((B, D), lambda b: (b, 0)),
            pl.BlockSpec((1, B, D), lambda b: (b // nbh, b % nbh, 0)),
            pl.BlockSpec((1, B, 16), lambda b: (b // nbh, b % nbh, 0)),
            pl.BlockSpec((D, D), lambda b: (0, 0)),
            pl.BlockSpec((1, D), lambda b: (0, 0)),
            pl.BlockSpec((D, D), lambda b: (0, 0)),
            pl.BlockSpec((1, D), lambda b: (0, 0)),
            pl.BlockSpec((2 * D, D), lambda b: (0, 0)),
            pl.BlockSpec((1, D), lambda b: (0, 0)),
        ],
        out_specs=pl.BlockSpec((B, D), lambda b: (b, 0)),
        out_shape=jax.ShapeDtypeStruct((N_NODES, D), jnp.float32),
    )(x, s2, c2, W_lin, b_lin, W_msg, b_msg, W_upd, b_upd)


def kernel(x, edge_index, W_lin, b_lin, W_msg, b_msg, W_upd, b_upd):
    row = edge_index[0].astype(jnp.int32)
    col = edge_index[1].astype(jnp.int32)
    s2, c2 = _sc_segment_sum(x, row, col)
    return _tc_dense(x, s2, c2, W_lin, b_lin.reshape(1, D), W_msg,
                     b_msg.reshape(1, D), W_upd, b_upd.reshape(1, D))


# speedup vs baseline: 2.1565x; 2.1565x over previous
import functools

import jax
import jax.numpy as jnp
from jax import lax
from jax.experimental import pallas as pl
from jax.experimental.pallas import tpu as pltpu
from jax.experimental.pallas import tpu_sc as plsc

N = 10000
E = 160000
D = 256
NZPAD = 64
RANGE = 320
ACC = 336
CH = 1280
NCH = E // CH
PB = 1408
RANK_BASE = 1


def _sc_body(xz_hbm, row_hbm, col_hbm, s_out, c_out,
             rbufA, cbufA, rbufB, cbufB, pend_loc, pend_col,
             acc, stgA, stgB, cnt, cntf, semA, semB, semG):
    c = lax.axis_index("c")
    s = lax.axis_index("s")
    w = c * 16 + s
    lo = w * RANGE
    iota = lax.iota(jnp.int32, 16)
    zero16 = jnp.zeros((16,), jnp.int32)

    def _fire_grp(g, stg):
        pltpu.async_copy(xz_hbm.at[pend_col.at[pl.ds(g * 16, 16)]], stg, semG)

    def _wait_grp(g, stg):
        pltpu.make_async_copy(
            xz_hbm.at[pend_col.at[pl.ds(g * 16, 16)]], stg, semG).wait()

    def _acc_grp(g, stg):
        loc16 = plsc.load_gather(pend_loc, [g * 16 + iota])
        rank16, last16 = plsc.scan_count(loc16)
        c16 = plsc.load_gather(cnt, [loc16])
        plsc.store_scatter(cnt, [loc16], c16 + rank16, mask=last16)
        def lb(l, _):
            lrow = plsc.load_gather(pend_loc,
                                    [jnp.full((16,), g * 16 + l, jnp.int32)])
            lsplat = jnp.full((16,), l, jnp.int32)
            for k in range(D // 16):
                v = plsc.load_gather(stg, [lsplat, 16 * k + iota])
                plsc.addupdate_scatter(acc, [lrow, 16 * k + iota], v)
            return 0
        lax.fori_loop(0, 16, lb, 0)

    def _consume(ng):
        _fire_grp(0, stgA)

        def pair(b, _):
            _fire_grp(2 * b + 1, stgB)
            _wait_grp(2 * b, stgA)
            _acc_grp(2 * b, stgA)
            _fire_grp(2 * b + 2, stgA)
            _wait_grp(2 * b + 1, stgB)
            _acc_grp(2 * b + 1, stgB)
            return 0
        lax.fori_loop(0, ng >> 1, pair, 0)
        _wait_grp(ng & ~1, stgA)

        @pl.when((ng & 1) == 1)
        def _():
            _acc_grp(ng & ~1, stgA)

    def _pad_pend(np_v):
        np_pad = (np_v + 15) & ~15
        idx16 = np_v + iota
        m = idx16 < np_pad
        plsc.store_scatter(pend_loc, [idx16], RANGE + iota, mask=m)
        plsc.store_scatter(pend_col, [idx16], N + iota, mask=m)
        return np_pad

    def _fire(ci, rbuf, cbuf, sem):
        off = ci * CH
        pltpu.async_copy(row_hbm.at[pl.ds(off, CH)], rbuf, sem)
        pltpu.async_copy(col_hbm.at[pl.ds(off, CH)], cbuf, sem)

    def _await(ci, rbuf, cbuf, sem):
        off = ci * CH
        pltpu.make_async_copy(row_hbm.at[pl.ds(off, CH)], rbuf, sem).wait()
        pltpu.make_async_copy(col_hbm.at[pl.ds(off, CH)], cbuf, sem).wait()

    def _scan_chunk(rbuf, cbuf, np_v):
        def vb(v, np_v):
            vidx = v * 16 + iota
            row16 = plsc.load_gather(rbuf, [vidx])
            col16 = plsc.load_gather(cbuf, [vidx])
            t16 = row16 - lo
            valid = (t16 >= 0) & (t16 < RANGE)
            cs = plsc.cumsum(jnp.where(valid, 1, 0))
            pc = plsc.all_reduce_population_count(valid)
            pos16 = np_v + cs - 1
            plsc.store_scatter(pend_loc, [pos16], t16, mask=valid)
            plsc.store_scatter(pend_col, [pos16], col16, mask=valid)
            return np_v + pc
        return lax.fori_loop(0, CH // 16, vb, np_v)

    def _process_chunk(rbuf, cbuf, np_v):
        np_v = _scan_chunk(rbuf, cbuf, np_v)
        ng = jnp.max(np_v) >> 4
        _consume(ng)
        ridx = (ng << 4) + iota
        loc16 = plsc.load_gather(pend_loc, [ridx])
        col16 = plsc.load_gather(pend_col, [ridx])
        rem_v = np_v & 15
        m = iota < rem_v
        plsc.store_scatter(pend_loc, [iota], jnp.where(m, loc16, RANGE + iota))
        plsc.store_scatter(pend_col, [iota], jnp.where(m, col16, N + iota))
        return rem_v

    for r in range(ACC // 16):
        cnt[pl.ds(16 * r, 16)] = zero16
    for r in range(PB // 16):
        pend_loc[pl.ds(16 * r, 16)] = RANGE + iota
        pend_col[pl.ds(16 * r, 16)] = N + iota
    for t in range(ACC // 16):
        pltpu.async_copy(xz_hbm.at[pend_col.at[pl.ds(0, 16)]],
                         acc.at[pl.ds(16 * t, 16)], semG).wait()

    np_v = zero16

    _fire(0, rbufA, cbufA, semA)

    def chunk_pair(j, np_v):
        _fire(2 * j + 1, rbufB, cbufB, semB)
        _await(2 * j, rbufA, cbufA, semA)
        np_v = _process_chunk(rbufA, cbufA, np_v)
        _fire(2 * j + 2, rbufA, cbufA, semA)
        _await(2 * j + 1, rbufB, cbufB, semB)
        np_v = _process_chunk(rbufB, cbufB, np_v)
        return np_v
    np_v = lax.fori_loop(0, (NCH - 1) // 2, chunk_pair, np_v)
    _await(NCH - 1, rbufA, cbufA, semA)
    np_v = _process_chunk(rbufA, cbufA, np_v)

    np_v = _pad_pend(np_v)
    _consume(jnp.max(np_v) >> 4)

    for k in range(RANGE // 16):
        cntf[pl.ds(16 * k, 16)] = cnt[pl.ds(16 * k, 16)].astype(jnp.float32)
    pltpu.sync_copy(acc.at[pl.ds(0, RANGE)], s_out.at[pl.ds(lo, RANGE)])
    pltpu.sync_copy(cntf, c_out.at[pl.ds(lo, RANGE)])


def _sc_segment_sum(xz, row, col):
    mesh = plsc.VectorSubcoreMesh(core_axis_name="c", subcore_axis_name="s")
    kern = functools.partial(
        pl.kernel,
        mesh=mesh,
        compiler_params=pltpu.CompilerParams(needs_layout_passes=False),
        out_type=[
            jax.ShapeDtypeStruct((32 * RANGE, D), jnp.float32),
            jax.ShapeDtypeStruct((32 * RANGE,), jnp.float32),
        ],
        scratch_types=[
            pltpu.VMEM((CH,), jnp.int32),
            pltpu.VMEM((CH,), jnp.int32),
            pltpu.VMEM((CH,), jnp.int32),
            pltpu.VMEM((CH,), jnp.int32),
            pltpu.VMEM((PB,), jnp.int32),
            pltpu.VMEM((PB,), jnp.int32),
            pltpu.VMEM((ACC, D), jnp.float32),
            pltpu.VMEM((16, D), jnp.float32),
            pltpu.VMEM((16, D), jnp.float32),
            pltpu.VMEM((ACC,), jnp.int32),
            pltpu.VMEM((RANGE,), jnp.float32),
            pltpu.SemaphoreType.DMA,
            pltpu.SemaphoreType.DMA,
            pltpu.SemaphoreType.DMA,
        ],
    )(_sc_body)
    return kern(xz, row, col)


def _tc_body(x_ref, s_ref, cnt_ref, wlin_ref, blin_ref, wmsg_ref, bmsg_ref,
             wupd_ref, bupd_ref, o_ref):
    xb = x_ref[...]
    hs = jnp.dot(xb, wlin_ref[...], preferred_element_type=jnp.float32) + blin_ref[...]
    cnt = cnt_ref[...]
    m = s_ref[...] / jnp.maximum(cnt, 1.0)
    hm = jnp.dot(m, wmsg_ref[...], preferred_element_type=jnp.float32) + bmsg_ref[...]
    ha = jnp.where(cnt > 0.0, hm, 0.0)
    h = (jnp.dot(hs, wupd_ref[0:D], preferred_element_type=jnp.float32)
         + jnp.dot(ha, wupd_ref[D:2 * D], preferred_element_type=jnp.float32)
         + bupd_ref[...])
    o_ref[...] = jnp.maximum(h, 0.0)


def _tc_dense(x, s2, cnt2, W_lin, b_lin, W_msg, b_msg, W_upd, b_upd):
    B = 1000
    return pl.pallas_call(
        _tc_body,
        grid=(N // B,),
        in_specs=[
            pl.BlockSpec((B, D), lambda b: (b, 0)),
            pl.BlockSpec((B, D), lambda b: (b, 0)),
            pl.BlockSpec((B, 1), lambda b: (b, 0)),
            pl.BlockSpec((D, D), lambda b: (0, 0)),
            pl.BlockSpec((1, D), lambda b: (0, 0)),
            pl.BlockSpec((D, D), lambda b: (0, 0)),
            pl.BlockSpec((1, D), lambda b: (0, 0)),
            pl.BlockSpec((2 * D, D), lambda b: (0, 0)),
            pl.BlockSpec((1, D), lambda b: (0, 0)),
        ],
        out_specs=pl.BlockSpec((B, D), lambda b: (b, 0)),
        out_shape=jax.ShapeDtypeStruct((N, D), jnp.float32),
    )(x, s2, cnt2, W_lin, b_lin, W_msg, b_msg, W_upd, b_upd)


def kernel(x, edge_index, W_lin, b_lin, W_msg, b_msg, W_upd, b_upd):
    row = edge_index[0].astype(jnp.int32)
    col = edge_index[1].astype(jnp.int32)
    xz = jnp.concatenate([x, jnp.zeros((NZPAD, D), jnp.float32)], axis=0)
    s2, c2 = _sc_segment_sum(xz, row, col)
    s2 = s2[:N]
    cnt2 = c2[:N].reshape(N, 1)
    return _tc_dense(x, s2, cnt2, W_lin, b_lin.reshape(1, D), W_msg,
                     b_msg.reshape(1, D), W_upd, b_upd.reshape(1, D))
